# baseline (speedup 1.0000x reference)
import functools

import jax
import jax.numpy as jnp
from jax import lax
from jax.experimental import pallas as pl
from jax.experimental.pallas import tpu as pltpu
from jax.experimental.pallas import tpu_sc as plsc

HIDDEN = 256
EDGE_HIDDEN = 16
N_NODES = 10000
N_EDGES = 160000

_NSC = 2
_NTILE = 16
_NW = _NSC * _NTILE
_WCOL = 128
_NPAD = 10240
_NSTRIPE = _NPAD // _NTILE

_EPW = N_EDGES // _NW
_GCH = 128
_GFULL = _EPW // _GCH
_GTAIL = _EPW - _GFULL * _GCH

_SCH = 80
_EPT = N_EDGES // _NTILE
_SROWS = _EPT // _SCH


def _sc_gather(node_emb, src):
    mesh = plsc.VectorSubcoreMesh(core_axis_name="c", subcore_axis_name="s")

    @functools.partial(
        pl.kernel,
        out_type=jax.ShapeDtypeStruct((N_EDGES, HIDDEN // 2), jnp.int32),
        mesh=mesh,
        scratch_types=[
            pltpu.VMEM((_EPW,), jnp.int32),
            pltpu.VMEM((_GCH, HIDDEN // 2), jnp.int32),
            pltpu.VMEM((_GCH, HIDDEN // 2), jnp.int32),
            pltpu.VMEM((_GTAIL, HIDDEN // 2), jnp.int32),
            pltpu.SemaphoreType.DMA,
            pltpu.SemaphoreType.DMA,
            pltpu.SemaphoreType.DMA,
            pltpu.SemaphoreType.DMA,
        ],
    )
    def k(table_hbm, src_hbm, out_hbm, idx_v, rows_a, rows_b, tail_v,
          sem_ga, sem_gb, sem_wa, sem_wb):
        c = lax.axis_index("c")
        s = lax.axis_index("s")
        wid = s * _NSC + c
        base = wid * _EPW
        pltpu.sync_copy(src_hbm.at[pl.ds(base, _EPW)], idx_v)

        def start_g(j, rows_ref, sem_g):
            pltpu.async_copy(
                table_hbm.at[idx_v.at[pl.ds(j * _GCH, _GCH)]], rows_ref, sem_g
            )

        def half(j, rows_ref, sem_g, sem_w):
            pltpu.make_async_copy(
                table_hbm.at[idx_v.at[pl.ds(0, _GCH)]], rows_ref, sem_g
            ).wait()
            pltpu.async_copy(
                rows_ref, out_hbm.at[pl.ds(base + j * _GCH, _GCH)], sem_w
            )
            pltpu.make_async_copy(
                rows_ref, out_hbm.at[pl.ds(base, _GCH)], sem_w
            ).wait()

            @pl.when(j + 2 <= _GFULL - 1)
            def _():
                start_g(j + 2, rows_ref, sem_g)

        start_g(0, rows_a, sem_ga)
        start_g(1, rows_b, sem_gb)

        def body(g, carry):
            half(2 * g, rows_a, sem_ga, sem_wa)
            half(2 * g + 1, rows_b, sem_gb, sem_wb)
            return carry

        lax.fori_loop(0, _GFULL // 2, body, 0)
        half(_GFULL - 1, rows_a, sem_ga, sem_wa)
        t0 = _GFULL * _GCH
        pltpu.async_copy(
            table_hbm.at[idx_v.at[pl.ds(t0, _GTAIL)]], tail_v, sem_ga
        ).wait()
        pltpu.sync_copy(tail_v, out_hbm.at[pl.ds(base + t0, _GTAIL)])

    return k(node_emb, src)


def _sc_scatter(msgs, dst, zrows):
    mesh = plsc.VectorSubcoreMesh(core_axis_name="c", subcore_axis_name="s")

    @functools.partial(
        pl.kernel,
        out_type=(
            jax.ShapeDtypeStruct((_NSC, _NPAD, _WCOL), jnp.float32),
            jax.ShapeDtypeStruct((_NPAD,), jnp.float32),
            jax.ShapeDtypeStruct((_NTILE, 1, _NPAD), jnp.float32),
        ),
        mesh=mesh,
        scratch_types=[
            pltpu.VMEM((_SCH,), jnp.int32),
            pltpu.VMEM((_SCH,), jnp.int32),
            pltpu.VMEM((_SCH, _WCOL), jnp.float32),
            pltpu.VMEM((_SCH, _WCOL), jnp.float32),
            pltpu.VMEM((1, _NPAD), jnp.float32),
            pltpu.VMEM((_NTILE, 1, _NSTRIPE), jnp.float32),
            pltpu.VMEM((_NSTRIPE,), jnp.float32),
            pltpu.VMEM_SHARED((_NPAD, _WCOL), jnp.float32),
            pltpu.SemaphoreType.DMA,
            pltpu.SemaphoreType.DMA,
            pltpu.SemaphoreType.DMA,
            pltpu.SemaphoreType.DMA,
        ],
        compiler_params=pltpu.CompilerParams(needs_layout_passes=False),
    )
    def k(msgs_hbm, dst_hbm, z_hbm, out_hbm, deg_hbm, stage_hbm, idx_a,
          idx_b, buf_a, buf_b, hist_v, red_v, res_v, acc_s, sem_ia, sem_ib,
          sem_ma, sem_mb):
        c = lax.axis_index("c")
        s = lax.axis_index("s")
        pltpu.sync_copy(z_hbm, acc_s.at[pl.ds(s * _NSTRIPE, _NSTRIPE)])

        @pl.when(c == 0)
        def _zero_hist():
            zv = jnp.zeros((16,), jnp.float32)

            def zbody(i, carry):
                hist_v[0, pl.ds(i * 16, 16)] = zv
                return carry

            lax.fori_loop(0, _NPAD // 16, zbody, 0)

        plsc.subcore_barrier()
        e0 = s * _EPT
        vones = jnp.full((16,), 1.0, jnp.float32)

        def start(j, idx_ref, buf_ref, sem_i, sem_m):
            pltpu.async_copy(dst_hbm.at[pl.ds(e0 + j * _SCH, _SCH)], idx_ref,
                             sem_i)
            pltpu.async_copy(msgs_hbm.at[c, pl.ds(e0 + j * _SCH, _SCH)],
                             buf_ref, sem_m)

        def fin(idx_ref, buf_ref, sem_i, sem_m):
            pltpu.make_async_copy(dst_hbm.at[pl.ds(e0, _SCH)], idx_ref,
                                  sem_i).wait()
            pltpu.make_async_copy(msgs_hbm.at[c, pl.ds(e0, _SCH)], buf_ref,
                                  sem_m).wait()
            pltpu.sync_copy(buf_ref, acc_s.at[idx_ref], add=True)

            @pl.when(c == 0)
            def _hist():
                for t in range(_SCH // 16):
                    idx16 = idx_ref[pl.ds(t * 16, 16)]
                    plsc.addupdate_scatter(hist_v.at[0], [idx16], vones)

        start(0, idx_a, buf_a, sem_ia, sem_ma)
        start(1, idx_b, buf_b, sem_ib, sem_mb)

        def body(g, carry):
            fin(idx_a, buf_a, sem_ia, sem_ma)
            start(2 * g + 2, idx_a, buf_a, sem_ia, sem_ma)
            fin(idx_b, buf_b, sem_ib, sem_mb)

            @pl.when(g < _SROWS // 2 - 1)
            def _():
                start(2 * g + 3, idx_b, buf_b, sem_ib, sem_mb)

            return carry

        lax.fori_loop(0, _SROWS // 2, body, 0)
        fin(idx_a, buf_a, sem_ia, sem_ma)
        plsc.subcore_barrier()
        pltpu.sync_copy(
            acc_s.at[pl.ds(s * _NSTRIPE, _NSTRIPE)],
            out_hbm.at[c, pl.ds(s * _NSTRIPE, _NSTRIPE)],
        )

        @pl.when(c == 0)
        def _deg_reduce():
            pltpu.sync_copy(hist_v, stage_hbm.at[s])
            plsc.subcore_barrier()
            pltpu.sync_copy(
                stage_hbm.at[:, :, pl.ds(s * _NSTRIPE, _NSTRIPE)], red_v
            )

            def rbody(g, carry):
                v = red_v[0, 0, pl.ds(g * 16, 16)]
                for t in range(1, _NTILE):
                    v = v + red_v[t, 0, pl.ds(g * 16, 16)]
                res_v[pl.ds(g * 16, 16)] = v
                return carry

            lax.fori_loop(0, _NSTRIPE // 16, rbody, 0)
            pltpu.sync_copy(res_v, deg_hbm.at[pl.ds(s * _NSTRIPE, _NSTRIPE)])

    return k(msgs, dst, zrows)


def _msg_body(eb_ref, sf_ref, w1_ref, b1_ref, w2_ref, b2_ref, out_ref):
    h = jnp.maximum(
        jnp.dot(eb_ref[...], w1_ref[...], preferred_element_type=jnp.float32)
        + b1_ref[...],
        0.0,
    )
    ew = jnp.dot(h.astype(jnp.bfloat16), w2_ref[...],
                 preferred_element_type=jnp.float32) + b2_ref[...]
    msg = ew * sf_ref[...].astype(jnp.float32)
    out_ref[0] = msg[:, :_WCOL]
    out_ref[1] = msg[:, _WCOL:]


def _tc_messages(edge_emb, sf, ew_W1, ew_b1, ew_W2, ew_b2):
    BE = 4000
    grid = N_EDGES // BE
    return pl.pallas_call(
        _msg_body,
        grid=(grid,),
        in_specs=[
            pl.BlockSpec((BE, EDGE_HIDDEN), lambda i: (i, 0)),
            pl.BlockSpec((BE, HIDDEN), lambda i: (i, 0)),
            pl.BlockSpec((EDGE_HIDDEN, HIDDEN), lambda i: (0, 0)),
            pl.BlockSpec((1, HIDDEN), lambda i: (0, 0)),
            pl.BlockSpec((HIDDEN, HIDDEN), lambda i: (0, 0)),
            pl.BlockSpec((1, HIDDEN), lambda i: (0, 0)),
        ],
        out_specs=pl.BlockSpec((_NSC, BE, _WCOL), lambda i: (0, i, 0)),
        out_shape=jax.ShapeDtypeStruct((_NSC, N_EDGES, _WCOL), jnp.float32),
    )(edge_emb.astype(jnp.bfloat16), sf, ew_W1.astype(jnp.bfloat16),
      ew_b1.reshape(1, -1), ew_W2.astype(jnp.bfloat16),
      ew_b2.reshape(1, -1))


def _final_body(x_ref, agg_ref, deg_ref, w1a_ref, w1b_ref, b1_ref, w2_ref,
                b2_ref, g_ref, beta_ref, out_ref):
    x = x_ref[...]
    deg = jnp.maximum(deg_ref[...], 1.0)
    a = jnp.concatenate([agg_ref[0], agg_ref[1]], axis=1) / deg
    h2 = jnp.maximum(
        jnp.dot(x, w1a_ref[...], preferred_element_type=jnp.float32)
        + jnp.dot(a, w1b_ref[...], preferred_element_type=jnp.float32)
        + b1_ref[...],
        0.0,
    )
    nu = jnp.dot(h2, w2_ref[...], preferred_element_type=jnp.float32) + b2_ref[...]
    y = x + nu
    mean = jnp.mean(y, axis=1, keepdims=True)
    yc = y - mean
    var = jnp.mean(yc * yc, axis=1, keepdims=True)
    out_ref[...] = yc * lax.rsqrt(var + 1e-5) * g_ref[...] + beta_ref[...]


def _tc_final(node_emb, agg, deg, nu_W1, nu_b1, nu_W2, nu_b2, ln_gamma,
              ln_beta):
    BN = 1000
    grid = N_NODES // BN
    return pl.pallas_call(
        _final_body,
        grid=(grid,),
        in_specs=[
            pl.BlockSpec((BN, HIDDEN), lambda i: (i, 0)),
            pl.BlockSpec((_NSC, BN, _WCOL), lambda i: (0, i, 0)),
            pl.BlockSpec((BN, 1), lambda i: (i, 0)),
            pl.BlockSpec((HIDDEN, HIDDEN), lambda i: (0, 0)),
            pl.BlockSpec((HIDDEN, HIDDEN), lambda i: (0, 0)),
            pl.BlockSpec((1, HIDDEN), lambda i: (0, 0)),
            pl.BlockSpec((HIDDEN, HIDDEN), lambda i: (0, 0)),
            pl.BlockSpec((1, HIDDEN), lambda i: (0, 0)),
            pl.BlockSpec((1, HIDDEN), lambda i: (0, 0)),
            pl.BlockSpec((1, HIDDEN), lambda i: (0, 0)),
        ],
        out_specs=pl.BlockSpec((BN, HIDDEN), lambda i: (i, 0)),
        out_shape=jax.ShapeDtypeStruct((N_NODES, HIDDEN), jnp.float32),
    )(node_emb, agg, deg, nu_W1[:HIDDEN], nu_W1[HIDDEN:],
      nu_b1.reshape(1, -1), nu_W2, nu_b2.reshape(1, -1),
      ln_gamma.reshape(1, -1), ln_beta.reshape(1, -1))


def kernel(node_emb, edge_index, edge_emb, ew_W1, ew_b1, ew_W2, ew_b2,
           nu_W1, nu_b1, nu_W2, nu_b2, ln_gamma, ln_beta):
    src = edge_index[0].astype(jnp.int32)
    dst = edge_index[1].astype(jnp.int32)
    zrows = jnp.zeros((_NSTRIPE, _WCOL), jnp.float32)
    node_bf = node_emb.astype(jnp.bfloat16)
    node_i32 = lax.bitcast_convert_type(
        node_bf.reshape(N_NODES, HIDDEN // 2, 2), jnp.int32
    )
    sf_i32 = _sc_gather(node_i32, src)
    sf = lax.bitcast_convert_type(sf_i32, jnp.bfloat16).reshape(
        N_EDGES, HIDDEN
    )
    msgs = _tc_messages(edge_emb, sf, ew_W1, ew_b1, ew_W2, ew_b2)
    agg, deg, _unused_stage = _sc_scatter(msgs, dst, zrows)
    deg2d = deg.reshape(_NPAD, 1)
    return _tc_final(node_emb, agg, deg2d, nu_W1, nu_b1, nu_W2, nu_b2,
                     ln_gamma, ln_beta)


# speedup vs baseline: 3.1046x; 3.1046x over previous
import functools

import jax
import jax.numpy as jnp
from jax import lax
from jax.experimental import pallas as pl
from jax.experimental.pallas import tpu as pltpu
from jax.experimental.pallas import tpu_sc as plsc

HIDDEN = 256
EDGE_HIDDEN = 16
N_NODES = 10000
N_EDGES = 160000

_NSC = 2
_NTILE = 16
_NW = _NSC * _NTILE
_WCOL = 128
_NPAD = 10240
_NSTRIPE = _NPAD // _NTILE

_EPW = N_EDGES // _NW
_GCH = 128
_GFULL = _EPW // _GCH
_GTAIL = _EPW - _GFULL * _GCH

_SCH = 80
_EPT = N_EDGES // _NTILE
_SROWS = _EPT // _SCH


def _sc_gather(node_emb, src):
    mesh = plsc.VectorSubcoreMesh(core_axis_name="c", subcore_axis_name="s")

    @functools.partial(
        pl.kernel,
        out_type=jax.ShapeDtypeStruct((N_EDGES, HIDDEN // 2), jnp.int32),
        mesh=mesh,
        scratch_types=[
            pltpu.VMEM((_EPW,), jnp.int32),
            pltpu.VMEM((_GCH, HIDDEN // 2), jnp.int32),
            pltpu.VMEM((_GCH, HIDDEN // 2), jnp.int32),
            pltpu.VMEM((_GTAIL, HIDDEN // 2), jnp.int32),
            pltpu.SemaphoreType.DMA,
            pltpu.SemaphoreType.DMA,
            pltpu.SemaphoreType.DMA,
            pltpu.SemaphoreType.DMA,
        ],
    )
    def k(table_hbm, src_hbm, out_hbm, idx_v, rows_a, rows_b, tail_v,
          sem_ga, sem_gb, sem_wa, sem_wb):
        c = lax.axis_index("c")
        s = lax.axis_index("s")
        wid = s * _NSC + c
        base = wid * _EPW
        pltpu.sync_copy(src_hbm.at[pl.ds(base, _EPW)], idx_v)

        def start_g(j, rows_ref, sem_g):
            pltpu.async_copy(
                table_hbm.at[idx_v.at[pl.ds(j * _GCH, _GCH)]], rows_ref, sem_g
            )

        def half(j, rows_ref, sem_g, sem_w):
            pltpu.make_async_copy(
                table_hbm.at[idx_v.at[pl.ds(0, _GCH)]], rows_ref, sem_g
            ).wait()
            pltpu.async_copy(
                rows_ref, out_hbm.at[pl.ds(base + j * _GCH, _GCH)], sem_w
            )
            pltpu.make_async_copy(
                rows_ref, out_hbm.at[pl.ds(base, _GCH)], sem_w
            ).wait()

            @pl.when(j + 2 <= _GFULL - 1)
            def _():
                start_g(j + 2, rows_ref, sem_g)

        start_g(0, rows_a, sem_ga)
        start_g(1, rows_b, sem_gb)

        def body(g, carry):
            half(2 * g, rows_a, sem_ga, sem_wa)
            half(2 * g + 1, rows_b, sem_gb, sem_wb)
            return carry

        lax.fori_loop(0, _GFULL // 2, body, 0)
        half(_GFULL - 1, rows_a, sem_ga, sem_wa)
        t0 = _GFULL * _GCH
        pltpu.async_copy(
            table_hbm.at[idx_v.at[pl.ds(t0, _GTAIL)]], tail_v, sem_ga
        ).wait()
        pltpu.sync_copy(tail_v, out_hbm.at[pl.ds(base + t0, _GTAIL)])

    return k(node_emb, src)


def _sc_scatter(msgs, dst, zrows):
    mesh = plsc.VectorSubcoreMesh(core_axis_name="c", subcore_axis_name="s")

    @functools.partial(
        pl.kernel,
        out_type=(
            jax.ShapeDtypeStruct((_NSC, _NPAD, _WCOL), jnp.float32),
            jax.ShapeDtypeStruct((_NPAD,), jnp.float32),
            jax.ShapeDtypeStruct((_NTILE, 1, _NPAD), jnp.float32),
        ),
        mesh=mesh,
        scratch_types=[
            pltpu.VMEM((_SCH,), jnp.int32),
            pltpu.VMEM((_SCH,), jnp.int32),
            pltpu.VMEM((_SCH, _WCOL), jnp.float32),
            pltpu.VMEM((_SCH, _WCOL), jnp.float32),
            pltpu.VMEM((1, _NPAD), jnp.float32),
            pltpu.VMEM((_NTILE, 1, _NSTRIPE), jnp.float32),
            pltpu.VMEM((_NSTRIPE,), jnp.float32),
            pltpu.VMEM_SHARED((_NPAD, _WCOL), jnp.float32),
            pltpu.SemaphoreType.DMA,
            pltpu.SemaphoreType.DMA,
            pltpu.SemaphoreType.DMA,
            pltpu.SemaphoreType.DMA,
        ],
        compiler_params=pltpu.CompilerParams(needs_layout_passes=False),
    )
    def k(msgs_hbm, dst_hbm, z_hbm, out_hbm, deg_hbm, stage_hbm, idx_a,
          idx_b, buf_a, buf_b, hist_v, red_v, res_v, acc_s, sem_ia, sem_ib,
          sem_ma, sem_mb):
        c = lax.axis_index("c")
        s = lax.axis_index("s")
        pltpu.sync_copy(z_hbm, acc_s.at[pl.ds(s * _NSTRIPE, _NSTRIPE)])

        @pl.when(c == 0)
        def _zero_hist():
            zv = jnp.zeros((16,), jnp.float32)

            def zbody(i, carry):
                hist_v[0, pl.ds(i * 16, 16)] = zv
                return carry

            lax.fori_loop(0, _NPAD // 16, zbody, 0)

        plsc.subcore_barrier()
        e0 = s * _EPT
        vones = jnp.full((16,), 1.0, jnp.float32)

        def start(j, idx_ref, buf_ref, sem_i, sem_m):
            pltpu.async_copy(dst_hbm.at[pl.ds(e0 + j * _SCH, _SCH)], idx_ref,
                             sem_i)
            pltpu.async_copy(msgs_hbm.at[c, pl.ds(e0 + j * _SCH, _SCH)],
                             buf_ref, sem_m)

        def fin(idx_ref, buf_ref, sem_i, sem_m):
            pltpu.make_async_copy(dst_hbm.at[pl.ds(e0, _SCH)], idx_ref,
                                  sem_i).wait()
            pltpu.make_async_copy(msgs_hbm.at[c, pl.ds(e0, _SCH)], buf_ref,
                                  sem_m).wait()
            pltpu.sync_copy(buf_ref, acc_s.at[idx_ref], add=True)

            @pl.when(c == 0)
            def _hist():
                for t in range(_SCH // 16):
                    idx16 = idx_ref[pl.ds(t * 16, 16)]
                    plsc.addupdate_scatter(hist_v.at[0], [idx16], vones)

        start(0, idx_a, buf_a, sem_ia, sem_ma)
        start(1, idx_b, buf_b, sem_ib, sem_mb)

        def body(g, carry):
            fin(idx_a, buf_a, sem_ia, sem_ma)
            start(2 * g + 2, idx_a, buf_a, sem_ia, sem_ma)
            fin(idx_b, buf_b, sem_ib, sem_mb)

            @pl.when(g < _SROWS // 2 - 1)
            def _():
                start(2 * g + 3, idx_b, buf_b, sem_ib, sem_mb)

            return carry

        lax.fori_loop(0, _SROWS // 2, body, 0)
        fin(idx_a, buf_a, sem_ia, sem_ma)
        plsc.subcore_barrier()
        pltpu.sync_copy(
            acc_s.at[pl.ds(s * _NSTRIPE, _NSTRIPE)],
            out_hbm.at[c, pl.ds(s * _NSTRIPE, _NSTRIPE)],
        )

        @pl.when(c == 0)
        def _deg_reduce():
            pltpu.sync_copy(hist_v, stage_hbm.at[s])
            plsc.subcore_barrier()
            pltpu.sync_copy(
                stage_hbm.at[:, :, pl.ds(s * _NSTRIPE, _NSTRIPE)], red_v
            )

            def rbody(g, carry):
                v = red_v[0, 0, pl.ds(g * 16, 16)]
                for t in range(1, _NTILE):
                    v = v + red_v[t, 0, pl.ds(g * 16, 16)]
                res_v[pl.ds(g * 16, 16)] = v
                return carry

            lax.fori_loop(0, _NSTRIPE // 16, rbody, 0)
            pltpu.sync_copy(res_v, deg_hbm.at[pl.ds(s * _NSTRIPE, _NSTRIPE)])

    return k(msgs, dst, zrows)


def _msg_body(eb_ref, sf_ref, w1_ref, b1_ref, w2_ref, b2_ref, out_ref):
    h = jnp.maximum(
        jnp.dot(eb_ref[...], w1_ref[...], preferred_element_type=jnp.float32)
        + b1_ref[...],
        0.0,
    )
    ew = jnp.dot(h.astype(jnp.bfloat16), w2_ref[...],
                 preferred_element_type=jnp.float32) + b2_ref[...]
    x = sf_ref[...]
    sf_lo = lax.bitcast_convert_type(x << 16, jnp.float32)
    sf_hi = lax.bitcast_convert_type(x & jnp.int32(-65536), jnp.float32)
    out_ref[0] = ew[:, :_WCOL] * sf_lo
    out_ref[1] = ew[:, _WCOL:] * sf_hi


def _tc_messages(edge_emb, sf, ew_W1, ew_b1, ew_W2, ew_b2):
    BE = 4000
    grid = N_EDGES // BE
    return pl.pallas_call(
        _msg_body,
        grid=(grid,),
        in_specs=[
            pl.BlockSpec((BE, EDGE_HIDDEN), lambda i: (i, 0)),
            pl.BlockSpec((BE, HIDDEN // 2), lambda i: (i, 0)),
            pl.BlockSpec((EDGE_HIDDEN, HIDDEN), lambda i: (0, 0)),
            pl.BlockSpec((1, HIDDEN), lambda i: (0, 0)),
            pl.BlockSpec((HIDDEN, HIDDEN), lambda i: (0, 0)),
            pl.BlockSpec((1, HIDDEN), lambda i: (0, 0)),
        ],
        out_specs=pl.BlockSpec((_NSC, BE, _WCOL), lambda i: (0, i, 0)),
        out_shape=jax.ShapeDtypeStruct((_NSC, N_EDGES, _WCOL), jnp.float32),
    )(edge_emb.astype(jnp.bfloat16), sf, ew_W1.astype(jnp.bfloat16),
      ew_b1.reshape(1, -1), ew_W2.astype(jnp.bfloat16),
      ew_b2.reshape(1, -1))


def _final_body(x_ref, agg_ref, deg_ref, w1a_ref, w1b_ref, b1_ref, w2_ref,
                b2_ref, g_ref, beta_ref, out_ref):
    x = x_ref[...]
    deg = jnp.maximum(deg_ref[...], 1.0)
    a = jnp.concatenate([agg_ref[0], agg_ref[1]], axis=1) / deg
    h2 = jnp.maximum(
        jnp.dot(x, w1a_ref[...], preferred_element_type=jnp.float32)
        + jnp.dot(a, w1b_ref[...], preferred_element_type=jnp.float32)
        + b1_ref[...],
        0.0,
    )
    nu = jnp.dot(h2, w2_ref[...], preferred_element_type=jnp.float32) + b2_ref[...]
    y = x + nu
    mean = jnp.mean(y, axis=1, keepdims=True)
    yc = y - mean
    var = jnp.mean(yc * yc, axis=1, keepdims=True)
    out_ref[...] = yc * lax.rsqrt(var + 1e-5) * g_ref[...] + beta_ref[...]


def _tc_final(node_emb, agg, deg, nu_W1, nu_b1, nu_W2, nu_b2, ln_gamma,
              ln_beta):
    BN = 1000
    grid = N_NODES // BN
    return pl.pallas_call(
        _final_body,
        grid=(grid,),
        in_specs=[
            pl.BlockSpec((BN, HIDDEN), lambda i: (i, 0)),
            pl.BlockSpec((_NSC, BN, _WCOL), lambda i: (0, i, 0)),
            pl.BlockSpec((BN, 1), lambda i: (i, 0)),
            pl.BlockSpec((HIDDEN, HIDDEN), lambda i: (0, 0)),
            pl.BlockSpec((HIDDEN, HIDDEN), lambda i: (0, 0)),
            pl.BlockSpec((1, HIDDEN), lambda i: (0, 0)),
            pl.BlockSpec((HIDDEN, HIDDEN), lambda i: (0, 0)),
            pl.BlockSpec((1, HIDDEN), lambda i: (0, 0)),
            pl.BlockSpec((1, HIDDEN), lambda i: (0, 0)),
            pl.BlockSpec((1, HIDDEN), lambda i: (0, 0)),
        ],
        out_specs=pl.BlockSpec((BN, HIDDEN), lambda i: (i, 0)),
        out_shape=jax.ShapeDtypeStruct((N_NODES, HIDDEN), jnp.float32),
    )(node_emb, agg, deg, nu_W1[:HIDDEN], nu_W1[HIDDEN:],
      nu_b1.reshape(1, -1), nu_W2, nu_b2.reshape(1, -1),
      ln_gamma.reshape(1, -1), ln_beta.reshape(1, -1))


def kernel(node_emb, edge_index, edge_emb, ew_W1, ew_b1, ew_W2, ew_b2,
           nu_W1, nu_b1, nu_W2, nu_b2, ln_gamma, ln_beta):
    src = edge_index[0].astype(jnp.int32)
    dst = edge_index[1].astype(jnp.int32)
    zrows = jnp.zeros((_NSTRIPE, _WCOL), jnp.float32)
    node_bf = node_emb.astype(jnp.bfloat16)
    lo = lax.bitcast_convert_type(node_bf[:, : HIDDEN // 2],
                                  jnp.uint16).astype(jnp.uint32)
    hi = lax.bitcast_convert_type(node_bf[:, HIDDEN // 2:],
                                  jnp.uint16).astype(jnp.uint32)
    node_i32 = lax.bitcast_convert_type(lo | (hi << 16), jnp.int32)
    sf_i32 = _sc_gather(node_i32, src)
    msgs = _tc_messages(edge_emb, sf_i32, ew_W1, ew_b1, ew_W2, ew_b2)
    agg, deg, _unused_stage = _sc_scatter(msgs, dst, zrows)
    deg2d = deg.reshape(_NPAD, 1)
    return _tc_final(node_emb, agg, deg2d, nu_W1, nu_b1, nu_W2, nu_b2,
                     ln_gamma, ln_beta)
